# baseline (speedup 1.0000x reference)
import functools

import jax
import jax.numpy as jnp
import numpy as np
from jax.experimental import pallas as pl

NQ = 10000
NT = 10000
E = 32000
D = 768
H = 4
DK = D // H
L = 2
R = 3
SQRT_DK = float(np.sqrt(DK))

TM = 2000
_PREC = jax.lax.Precision.HIGHEST


def _fold_body(a_ref, b_ref, o_ref):
    o_ref[0] = jax.lax.dot_general(
        a_ref[0], b_ref[0], (((1,), (0,)), ((), ())),
        preferred_element_type=jnp.float32, precision=_PREC)


def _fold_mm(A, B):
    G = A.shape[0]
    return pl.pallas_call(
        _fold_body,
        grid=(G,),
        in_specs=[pl.BlockSpec((1, D, D), lambda g: (g, 0, 0)),
                  pl.BlockSpec((1, D, D), lambda g: (g, 0, 0))],
        out_specs=pl.BlockSpec((1, D, D), lambda g: (g, 0, 0)),
        out_shape=jax.ShapeDtypeStruct((G, D, D), jnp.float32),
    )(A, B)


def _proj_body(a_ref, b_ref, bias_ref, o_ref):
    h = pl.program_id(2)
    o_ref[0, 0] = jax.lax.dot_general(
        a_ref[...], b_ref[0, :, h, :], (((1,), (0,)), ((), ())),
        preferred_element_type=jnp.float32, precision=_PREC) + bias_ref[0, h, :]


def _proj_mm(A, B, bias):
    G = B.shape[0]
    M = A.shape[0]
    Mt = M // TM
    Bh = B.reshape(G, D, H, DK)
    biash = bias.reshape(G, H, DK)
    return pl.pallas_call(
        _proj_body,
        grid=(Mt, G, H),
        in_specs=[pl.BlockSpec((TM, D), lambda i, g, h: (i, 0)),
                  pl.BlockSpec((1, D, H, DK), lambda i, g, h: (g, 0, 0, 0)),
                  pl.BlockSpec((1, H, DK), lambda i, g, h: (g, 0, 0))],
        out_specs=pl.BlockSpec((1, 1, TM, DK), lambda i, g, h: (g, h, i, 0)),
        out_shape=jax.ShapeDtypeStruct((G, H, M, DK), jnp.float32),
    )(A, Bh, biash)


def _out_body(a_ref, b_ref, bias_ref, o_ref):
    o_ref[...] = jax.lax.dot_general(
        a_ref[...], b_ref[...], (((1,), (0,)), ((), ())),
        preferred_element_type=jnp.float32, precision=_PREC) + bias_ref[...]


def _out_mm(A, B, bias):
    M = A.shape[0]
    TMO = 1000
    Mt = M // TMO
    return pl.pallas_call(
        _out_body,
        grid=(Mt,),
        in_specs=[pl.BlockSpec((TMO, D), lambda i: (i, 0)),
                  pl.BlockSpec((D, D), lambda i: (0, 0)),
                  pl.BlockSpec((D,), lambda i: (0,))],
        out_specs=pl.BlockSpec((TMO, D), lambda i: (i, 0)),
        out_shape=jax.ShapeDtypeStruct((M, D), jnp.float32),
    )(A, B, bias)


def _block_diag(rel):
    G = rel.shape[0]
    bd = jnp.zeros((G, H, DK, H, DK), jnp.float32)
    ih = jnp.arange(H)
    bd = bd.at[:, ih, :, ih, :].set(rel.transpose(1, 0, 2, 3))
    return bd.reshape(G, D, D)


def _edge_agg(ktab, vtab, qtab, src, dst, ndst):
    ke = ktab[:, src, :]
    qe = qtab[:, dst, :]
    t = jnp.sum(qe * ke, axis=-1)
    p = jnp.exp(t)
    s = jax.ops.segment_sum(p.T, dst, num_segments=ndst)
    m = vtab[:, src, :] * p[:, :, None]
    aggu = jax.ops.segment_sum(m.transpose(1, 0, 2), dst, num_segments=ndst)
    agg = aggu / (s[:, :, None] + 1e-9)
    return agg.reshape(ndst, D)


def kernel(h_query, h_tag, give_src, give_dst, inter1_src, inter1_dst,
           inter2_src, inter2_dst, Wk, bk, Wq, bq, Wv, bv, rel_att, rel_msg,
           rel_pri, Wout, bout):
    scale = (rel_pri / SQRT_DK)[:, :, :, None, None]
    att_s = (rel_att * scale).reshape(L * R, H, DK, DK)
    msg = rel_msg.reshape(L * R, H, DK, DK)
    bd_att = _block_diag(att_s)
    bd_msg = _block_diag(msg)
    stype = jnp.array([0, 1, 1], jnp.int32)
    WkT = jnp.transpose(Wk, (0, 1, 3, 2))[:, stype]
    WvT = jnp.transpose(Wv, (0, 1, 3, 2))[:, stype]
    A = jnp.concatenate([WkT.reshape(L * R, D, D),
                         WvT.reshape(L * R, D, D)], axis=0)
    BD = jnp.concatenate([bd_att, bd_msg], axis=0)
    folded = _fold_mm(A, BD)
    WKf = folded[:L * R].reshape(L, R, D, D)
    WVf = folded[L * R:].reshape(L, R, D, D)
    bkr = bk[:, stype]
    bvr = bv[:, stype]
    bKf = jnp.einsum('lrd,lrde->lre', bkr, bd_att.reshape(L, R, D, D))
    bVf = jnp.einsum('lrd,lrde->lre', bvr, bd_msg.reshape(L, R, D, D))
    WqT = jnp.transpose(Wq, (0, 1, 3, 2))

    hq, ht = h_query, h_tag
    for l in range(L):
        Bq = jnp.stack([WKf[l, 0], WVf[l, 0], WqT[l, 0]], axis=0)
        biasq = jnp.stack([bKf[l, 0], bVf[l, 0], bq[l, 0]], axis=0)
        Pq = _proj_mm(hq, Bq, biasq)
        Bt = jnp.stack([WKf[l, 1], WKf[l, 2], WVf[l, 1], WVf[l, 2],
                        WqT[l, 1]], axis=0)
        biast = jnp.stack([bKf[l, 1], bKf[l, 2], bVf[l, 1], bVf[l, 2],
                           bq[l, 1]], axis=0)
        Pt = _proj_mm(ht, Bt, biast)

        agg_t0 = _edge_agg(Pq[0], Pq[1], Pt[4], give_src, give_dst, NT)
        agg_q = _edge_agg(Pt[0], Pt[2], Pq[2], inter1_src, inter1_dst, NQ)
        agg_t2 = _edge_agg(Pt[1], Pt[3], Pt[4], inter2_src, inter2_dst, NT)

        hq = agg_q + hq
        ht = 0.5 * (agg_t0 + agg_t2) + ht

    out_q = _out_mm(hq, Wout.T, bout)
    out_t = _out_mm(ht, Wout.T, bout)
    return (out_q, out_t)


# speedup vs baseline: 2.8479x; 2.8479x over previous
import functools

import jax
import jax.numpy as jnp
import numpy as np
from jax import lax
from jax.experimental import pallas as pl
from jax.experimental.pallas import tpu as pltpu
from jax.experimental.pallas import tpu_sc as plsc

NQ = 10000
NT = 10000
E = 32000
D = 768
H = 4
DK = D // H
L = 2
R = 3
SQRT_DK = float(np.sqrt(DK))

TM = 2000
_PREC = jax.lax.Precision.HIGHEST


def _fold_body(a_ref, b_ref, o_ref):
    o_ref[0] = jax.lax.dot_general(
        a_ref[0], b_ref[0], (((1,), (0,)), ((), ())),
        preferred_element_type=jnp.float32, precision=_PREC)


def _fold_mm(A, B):
    G = A.shape[0]
    return pl.pallas_call(
        _fold_body,
        grid=(G,),
        in_specs=[pl.BlockSpec((1, D, D), lambda g: (g, 0, 0)),
                  pl.BlockSpec((1, D, D), lambda g: (g, 0, 0))],
        out_specs=pl.BlockSpec((1, D, D), lambda g: (g, 0, 0)),
        out_shape=jax.ShapeDtypeStruct((G, D, D), jnp.float32),
    )(A, B)


def _proj_body(a_ref, b_ref, bias_ref, o_ref):
    h = pl.program_id(2)
    acc = bias_ref[0, h, :][None, :]
    for kh in range(H):
        acc = acc + jax.lax.dot_general(
            a_ref[kh], b_ref[0, pl.ds(kh * DK, DK), h, :],
            (((1,), (0,)), ((), ())),
            preferred_element_type=jnp.float32, precision=_PREC)
    o_ref[0, 0] = acc


def _proj_mm(A, B, bias, splitg):
    G = B.shape[0]
    Mt = NQ // TM
    Bh = B.reshape(G, D, H, DK)
    biash = bias.reshape(G, H, DK)

    def a_map(i, g, h):
        return (0, jnp.where(g >= splitg, i + Mt, i), 0)

    return pl.pallas_call(
        _proj_body,
        grid=(Mt, G, H),
        in_specs=[pl.BlockSpec((H, TM, DK), a_map),
                  pl.BlockSpec((1, D, H, DK), lambda i, g, h: (g, 0, 0, 0)),
                  pl.BlockSpec((1, H, DK), lambda i, g, h: (g, 0, 0))],
        out_specs=pl.BlockSpec((1, 1, TM, DK), lambda i, g, h: (g, h, i, 0)),
        out_shape=jax.ShapeDtypeStruct((G, H, NQ, DK), jnp.float32),
    )(A, Bh, biash)


def _projh_body(a_ref, b_ref, bias_ref, o_ref):
    h = pl.program_id(2)
    hh = pl.program_id(3)
    acc = bias_ref[0, h, hh, :][None, :]
    for kh in range(H):
        acc = acc + jax.lax.dot_general(
            a_ref[kh], b_ref[0, pl.ds(kh * DK, DK), h, hh, :],
            (((1,), (0,)), ((), ())),
            preferred_element_type=jnp.float32, precision=_PREC)
    o_ref[0, 0, 0] = acc


def _proj_mm_half(A, B, bias, splitg):
    G = B.shape[0]
    Mt = NQ // TM
    Bh = B.reshape(G, D, H, 2, VH)
    biash = bias.reshape(G, H, 2, VH)

    def a_map(i, g, h, hh):
        return (0, jnp.where(g >= splitg, i + Mt, i), 0)

    return pl.pallas_call(
        _projh_body,
        grid=(Mt, G, H, 2),
        in_specs=[pl.BlockSpec((H, TM, DK), a_map),
                  pl.BlockSpec((1, D, H, 2, VH),
                               lambda i, g, h, hh: (g, 0, 0, 0, 0)),
                  pl.BlockSpec((1, H, 2, VH),
                               lambda i, g, h, hh: (g, 0, 0, 0))],
        out_specs=pl.BlockSpec((1, 1, 1, TM, 96),
                               lambda i, g, h, hh: (g, h, hh, i, 0)),
        out_shape=jax.ShapeDtypeStruct((G, H, 2, NQ, 96), jnp.float32),
    )(A, Bh, biash)


def _out_body(a_ref, b_ref, bias_ref, o_ref):
    acc = bias_ref[...][None, :]
    for kh in range(H):
        acc = acc + jax.lax.dot_general(
            a_ref[kh], b_ref[pl.ds(kh * DK, DK), :], (((1,), (0,)), ((), ())),
            preferred_element_type=jnp.float32, precision=_PREC)
    o_ref[...] = acc


def _out_mm(A, B, bias, rofsb):
    TMO = 1000
    Mt = NQ // TMO
    return pl.pallas_call(
        _out_body,
        grid=(Mt,),
        in_specs=[pl.BlockSpec((H, TMO, DK), lambda i: (0, i + rofsb, 0)),
                  pl.BlockSpec((D, D), lambda i: (0, 0)),
                  pl.BlockSpec((D,), lambda i: (0,))],
        out_specs=pl.BlockSpec((TMO, D), lambda i: (i, 0)),
        out_shape=jax.ShapeDtypeStruct((NQ, D), jnp.float32),
    )(A, B, bias)


def _block_diag(rel):
    G = rel.shape[0]
    bd = jnp.zeros((G, H, DK, H, DK), jnp.float32)
    ih = jnp.arange(H)
    bd = bd.at[:, ih, :, ih, :].set(rel.transpose(1, 0, 2, 3))
    return bd.reshape(G, D, D)


EC = 80
EPT = E // 16
NCH = EPT // EC
RPT = 640
RW = 80
VH = 96
AW = 112

_i32 = jnp.int32


N2 = NQ + NT
RPT2 = 1280


def _sc_layer_kernel(TKQ, TV, h_all, esrc, edst, zrows,
                     new_h,
                     agg_sh, idx_s, idx_d, idx_g,
                     kbuf, qbuf, vbuf, mbuf, ptile):
    c = lax.axis_index("c")
    tid = lax.axis_index("s")
    iota = lax.iota(_i32, 16)
    fone = jnp.float32(1.0)
    fzero = jnp.float32(0.0)
    e0mask = jnp.where(iota == 0, fone, fzero)
    onehot = [jnp.where(iota == j2, fone, fzero) for j2 in range(16)]

    nblocks = jnp.minimum(jnp.int32(NT) - tid * RPT, RPT) // RW

    nb2 = jnp.minimum(jnp.int32(N2) - tid * RPT2, RPT2) // RW
    for hl4 in range(2):
        h4 = 2 * c + hl4

        def pib(b, _):
            rs = pl.ds(pl.multiple_of(tid * RPT2 + b * RW, 8), RW)
            pltpu.sync_copy(h_all.at[h4, rs], kbuf)
            pltpu.sync_copy(kbuf, new_h.at[h4, rs])
            return 0
        lax.fori_loop(0, nb2, pib, 0)
    plsc.subcore_barrier()

    def zero_acc():
        def zb(b, _):
            rs = pl.ds(pl.multiple_of(tid * RPT + b * RW, 8), RW)
            pltpu.sync_copy(zrows, agg_sh.at[rs])
            return 0
        lax.fori_loop(0, nblocks, zb, 0)

    def load_idx(eofs, ci):
        eb = pl.multiple_of(eofs + tid * EPT + ci * EC, 8)
        pltpu.sync_copy(esrc.at[pl.ds(eb, EC)], idx_s)
        pltpu.sync_copy(edst.at[pl.ds(eb, EC)], idx_d)

    def offset_gather(base_idx, base, tab, buf):
        for u in range(5):
            sl = pl.ds(u * 16, 16)
            idx_g[sl] = base_idx[sl] + base
        pltpu.sync_copy(tab.at[idx_g], buf)

    def round0(eofs, kbase, qbase, vbase):
        def chunk(ci, _):
            load_idx(eofs, ci)
            offset_gather(idx_s, kbase, TKQ, kbuf)
            offset_gather(idx_d, qbase, TKQ, qbuf)
            offset_gather(idx_s, vbase, TV, vbuf)

            def grp(g2, _g):
                pacc = jnp.zeros((16,), jnp.float32)
                for j2 in range(16):
                    j = g2 * 16 + j2
                    acc = jnp.zeros((16,), jnp.float32)
                    for u in range(12):
                        sl = pl.ds(u * 16, 16)
                        acc = acc + qbuf[j, sl] * kbuf[j, sl]
                    pv = jnp.exp(jnp.full((16,), jnp.sum(acc), jnp.float32))
                    pacc = pacc + pv * onehot[j2]
                    for u in range(6):
                        sl = pl.ds(u * 16, 16)
                        mbuf[j, sl] = vbuf[j, sl] * pv
                    mbuf[j, pl.ds(VH, 16)] = pv * e0mask
                ptile[pl.ds((ci * 5 + g2) * 16, 16)] = pacc
                return 0

            lax.fori_loop(0, 5, grp, 0)
            pltpu.sync_copy(mbuf, agg_sh.at[idx_d], add=True)
            return 0

        lax.fori_loop(0, NCH, chunk, 0)

    def round1(eofs, vbase):
        def chunk(ci, _):
            load_idx(eofs, ci)
            offset_gather(idx_s, vbase, TV, vbuf)

            def grp(g2, _g):
                pvec = ptile[pl.ds((ci * 5 + g2) * 16, 16)]
                for j2 in range(16):
                    j = g2 * 16 + j2
                    pv = jnp.full((16,), pvec[j2], jnp.float32)
                    for u in range(6):
                        sl = pl.ds(u * 16, 16)
                        mbuf[j, sl] = vbuf[j, sl] * pv
                    mbuf[j, pl.ds(VH, 16)] = pv * e0mask
                return 0

            lax.fori_loop(0, 5, grp, 0)
            pltpu.sync_copy(mbuf, agg_sh.at[idx_d], add=True)
            return 0

        lax.fori_loop(0, NCH, chunk, 0)

    def finalize(head, nodeofs, half, scale):
        hs = pl.ds(half * VH, VH)

        def fb(b, _):
            r0 = pl.multiple_of(tid * RPT + b * RW, 8)
            rs = pl.ds(r0, RW)
            rs2 = pl.ds(pl.multiple_of(nodeofs + r0, 8), RW)
            pltpu.sync_copy(agg_sh.at[rs], mbuf)
            pltpu.sync_copy(new_h.at[head, rs2, hs], vbuf)

            def frow(j, _f):
                sv = jnp.full((16,), mbuf[j, pl.ds(VH, 16)][0], jnp.float32)
                rv = jnp.full((16,), scale, jnp.float32) / (sv + 1e-9)
                for u in range(6):
                    sl = pl.ds(u * 16, 16)
                    vbuf[j, sl] = mbuf[j, sl] * rv + vbuf[j, sl]
                return 0

            lax.fori_loop(0, RW, frow, 0)
            pltpu.sync_copy(vbuf, new_h.at[head, rs2, hs])
            return 0

        lax.fori_loop(0, nblocks, fb, 0)

    def rel_body(ri, _):
        hl = ri // 3
        rr = ri - hl * 3
        head = 2 * c + hl
        n = jnp.int32(NQ)
        toff = 2 * H * NQ
        kbase = jnp.where(rr == 0, toff + head * n,
                          jnp.where(rr == 1, head * n,
                                    toff + (H + head) * n))
        qbase = jnp.where(rr == 0, (H + head) * n, toff + (2 * H + head) * n)
        voff = 2 * H * NQ
        vbase = jnp.where(rr == 0, voff + head * 2 * n,
                          jnp.where(rr == 1, head * 2 * n,
                                    voff + (H + head) * 2 * n))
        eofs = jnp.where(rr == 0, E, jnp.where(rr == 1, 0, 2 * E))
        nodeofs = jnp.where(rr == 0, 0, NQ)
        scale = jnp.where(rr == 0, fone, jnp.float32(0.5))

        zero_acc()
        plsc.subcore_barrier()
        round0(eofs, kbase, qbase, vbase)
        plsc.subcore_barrier()
        finalize(head, nodeofs, 0, scale)
        zero_acc()
        plsc.subcore_barrier()
        round1(eofs, vbase + n)
        plsc.subcore_barrier()
        finalize(head, nodeofs, 1, scale)
        return 0

    lax.fori_loop(0, 6, rel_body, 0)


_SC_SCRATCH = [
    pltpu.VMEM_SHARED((NT, AW), jnp.float32),
    pltpu.VMEM((EC,), _i32),
    pltpu.VMEM((EC,), _i32),
    pltpu.VMEM((EC,), _i32),
    pltpu.VMEM((EC, DK), jnp.float32),
    pltpu.VMEM((EC, DK), jnp.float32),
    pltpu.VMEM((EC, VH), jnp.float32),
    pltpu.VMEM((EC, AW), jnp.float32),
    pltpu.VMEM((EPT,), jnp.float32),
]


def _sc_layer(TKQ, TV, h_all, esrc, edst):
    zrows = jnp.zeros((RW, AW), jnp.float32)
    fn = pl.kernel(
        _sc_layer_kernel,
        out_type=jax.ShapeDtypeStruct((H, N2, DK), jnp.float32),
        mesh=plsc.VectorSubcoreMesh(core_axis_name="c", subcore_axis_name="s"),
        scratch_types=_SC_SCRATCH,
        compiler_params=pltpu.CompilerParams(use_tc_tiling_on_sc=False,
                                             needs_layout_passes=False),
    )
    return fn(TKQ, TV, h_all, esrc, edst, zrows)


def kernel(h_query, h_tag, give_src, give_dst, inter1_src, inter1_dst,
           inter2_src, inter2_dst, Wk, bk, Wq, bq, Wv, bv, rel_att, rel_msg,
           rel_pri, Wout, bout):
    scale = (rel_pri / SQRT_DK)[:, :, :, None, None]
    att_s = (rel_att * scale).reshape(L * R, H, DK, DK)
    msg = rel_msg.reshape(L * R, H, DK, DK)
    bd_att = _block_diag(att_s)
    bd_msg = _block_diag(msg)
    stype = jnp.array([0, 1, 1], jnp.int32)
    WkT = jnp.transpose(Wk, (0, 1, 3, 2))[:, stype]
    WvT = jnp.transpose(Wv, (0, 1, 3, 2))[:, stype]
    A = jnp.concatenate([WkT.reshape(L * R, D, D),
                         WvT.reshape(L * R, D, D)], axis=0)
    BD = jnp.concatenate([bd_att, bd_msg], axis=0)
    folded = _fold_mm(A, BD)
    WKf = folded[:L * R].reshape(L, R, D, D)
    WVf = folded[L * R:].reshape(L, R, D, D)
    bkr = bk[:, stype]
    bvr = bv[:, stype]
    bKf = jnp.einsum('lrd,lrde->lre', bkr, bd_att.reshape(L, R, D, D))
    bVf = jnp.einsum('lrd,lrde->lre', bvr, bd_msg.reshape(L, R, D, D))
    WqT = jnp.transpose(Wq, (0, 1, 3, 2))

    h_all = jnp.concatenate(
        [h_query.reshape(NQ, H, DK).transpose(1, 0, 2),
         h_tag.reshape(NT, H, DK).transpose(1, 0, 2)], axis=1)
    esrc = jnp.concatenate([give_src, inter1_src, inter2_src]).astype(_i32)
    edst = jnp.concatenate([give_dst, inter1_dst, inter2_dst]).astype(_i32)

    for l in range(L):
        Pkq = _proj_mm(
            h_all,
            jnp.stack([WKf[l, 0], WqT[l, 0], WKf[l, 1], WKf[l, 2], WqT[l, 1]]),
            jnp.stack([bKf[l, 0], bq[l, 0], bKf[l, 1], bKf[l, 2], bq[l, 1]]),
            splitg=2)
        Pv = _proj_mm_half(
            h_all,
            jnp.stack([WVf[l, 0], WVf[l, 1], WVf[l, 2]]),
            jnp.stack([bVf[l, 0], bVf[l, 1], bVf[l, 2]]),
            splitg=1)

        h_all = _sc_layer(Pkq.reshape(5 * H * NQ, DK),
                          Pv.reshape(3 * H * 2 * NQ, VH),
                          h_all, esrc, edst)

    out_q = _out_mm(h_all, Wout.T, bout, 0)
    out_t = _out_mm(h_all, Wout.T, bout, NQ // 1000)
    return (out_q, out_t)


# speedup vs baseline: 4.1866x; 1.4701x over previous
import functools

import jax
import jax.numpy as jnp
import numpy as np
from jax import lax
from jax.experimental import pallas as pl
from jax.experimental.pallas import tpu as pltpu
from jax.experimental.pallas import tpu_sc as plsc

NQ = 10000
NT = 10000
E = 32000
D = 768
H = 4
DK = D // H
L = 2
R = 3
SQRT_DK = float(np.sqrt(DK))

TM = 2000
_PREC = jax.lax.Precision.DEFAULT


def _fold_body(a_ref, b_ref, o_ref):
    o_ref[0] = jax.lax.dot_general(
        a_ref[0], b_ref[0], (((1,), (0,)), ((), ())),
        preferred_element_type=jnp.float32, precision=_PREC)


def _fold_mm(A, B):
    G = A.shape[0]
    return pl.pallas_call(
        _fold_body,
        grid=(G,),
        in_specs=[pl.BlockSpec((1, D, D), lambda g: (g, 0, 0)),
                  pl.BlockSpec((1, D, D), lambda g: (g, 0, 0))],
        out_specs=pl.BlockSpec((1, D, D), lambda g: (g, 0, 0)),
        out_shape=jax.ShapeDtypeStruct((G, D, D), jnp.float32),
    )(A, B)


def _proj_body(a_ref, b_ref, bias_ref, o_ref):
    h = pl.program_id(2)
    acc = bias_ref[0, h, :][None, :]
    for kh in range(H):
        acc = acc + jax.lax.dot_general(
            a_ref[kh], b_ref[0, pl.ds(kh * DK, DK), h, :],
            (((1,), (0,)), ((), ())),
            preferred_element_type=jnp.float32, precision=_PREC)
    o_ref[0, 0] = acc


def _proj_mm(A, B, bias, splitg):
    G = B.shape[0]
    Mt = NQ // TM
    Bh = B.reshape(G, D, H, DK)
    biash = bias.reshape(G, H, DK)

    def a_map(i, g, h):
        return (0, jnp.where(g >= splitg, i + Mt, i), 0)

    return pl.pallas_call(
        _proj_body,
        grid=(Mt, G, H),
        in_specs=[pl.BlockSpec((H, TM, DK), a_map),
                  pl.BlockSpec((1, D, H, DK), lambda i, g, h: (g, 0, 0, 0)),
                  pl.BlockSpec((1, H, DK), lambda i, g, h: (g, 0, 0))],
        out_specs=pl.BlockSpec((1, 1, TM, DK), lambda i, g, h: (g, h, i, 0)),
        out_shape=jax.ShapeDtypeStruct((G, H, NQ, DK), jnp.float32),
    )(A, Bh, biash)


def _projh_body(a_ref, b_ref, bias_ref, o_ref):
    h = pl.program_id(2)
    hh = pl.program_id(3)
    acc = bias_ref[0, h, hh, :][None, :]
    for kh in range(H):
        acc = acc + jax.lax.dot_general(
            a_ref[kh], b_ref[0, pl.ds(kh * DK, DK), h, hh, :],
            (((1,), (0,)), ((), ())),
            preferred_element_type=jnp.float32, precision=_PREC)
    o_ref[0, 0, 0] = acc


def _proj_mm_half(A, B, bias, splitg):
    G = B.shape[0]
    Mt = NQ // TM
    Bh = B.reshape(G, D, H, 2, VH)
    biash = bias.reshape(G, H, 2, VH)

    def a_map(i, g, h, hh):
        return (0, jnp.where(g >= splitg, i + Mt, i), 0)

    return pl.pallas_call(
        _projh_body,
        grid=(Mt, G, H, 2),
        in_specs=[pl.BlockSpec((H, TM, DK), a_map),
                  pl.BlockSpec((1, D, H, 2, VH),
                               lambda i, g, h, hh: (g, 0, 0, 0, 0)),
                  pl.BlockSpec((1, H, 2, VH),
                               lambda i, g, h, hh: (g, 0, 0, 0))],
        out_specs=pl.BlockSpec((1, 1, 1, TM, 96),
                               lambda i, g, h, hh: (g, h, hh, i, 0)),
        out_shape=jax.ShapeDtypeStruct((G, H, 2, NQ, 96), jnp.float32),
    )(A, Bh, biash)


def _out_body(a_ref, b_ref, bias_ref, o_ref):
    acc = bias_ref[...][None, :]
    for kh in range(H):
        acc = acc + jax.lax.dot_general(
            a_ref[kh], b_ref[pl.ds(kh * DK, DK), :], (((1,), (0,)), ((), ())),
            preferred_element_type=jnp.float32, precision=_PREC)
    o_ref[...] = acc


def _out_mm(A, B, bias, rofsb):
    TMO = 1000
    Mt = NQ // TMO
    return pl.pallas_call(
        _out_body,
        grid=(Mt,),
        in_specs=[pl.BlockSpec((H, TMO, DK), lambda i: (0, i + rofsb, 0)),
                  pl.BlockSpec((D, D), lambda i: (0, 0)),
                  pl.BlockSpec((D,), lambda i: (0,))],
        out_specs=pl.BlockSpec((TMO, D), lambda i: (i, 0)),
        out_shape=jax.ShapeDtypeStruct((NQ, D), jnp.float32),
    )(A, B, bias)


def _block_diag(rel):
    G = rel.shape[0]
    bd = jnp.zeros((G, H, DK, H, DK), jnp.float32)
    ih = jnp.arange(H)
    bd = bd.at[:, ih, :, ih, :].set(rel.transpose(1, 0, 2, 3))
    return bd.reshape(G, D, D)


EC = 80
EPT = E // 16
NCH = EPT // EC
RPT = 640
RW = 80
VH = 96
AW = 112

_i32 = jnp.int32


N2 = NQ + NT
RPT2 = 1280


def _sc_layer_kernel(TKQ, TV, h_all, esrc, edst, zrows,
                     new_h,
                     agg_sh, idx_s, idx_d, idx_k, idx_q, idx_v, idx_sc,
                     kbuf, qbuf, vbuf, mbuf, ptile,
                     sis, sid, sgk, sgq, sgv):
    c = lax.axis_index("c")
    tid = lax.axis_index("s")
    iota = lax.iota(_i32, 16)
    fone = jnp.float32(1.0)
    fzero = jnp.float32(0.0)
    e0mask = jnp.where(iota == 0, fone, fzero)
    onehot = [jnp.where(iota == j2, fone, fzero) for j2 in range(16)]

    nblocks = jnp.minimum(jnp.int32(NT) - tid * RPT, RPT) // RW

    nb2 = jnp.minimum(jnp.int32(N2) - tid * RPT2, RPT2) // RW
    for hl4 in range(2):
        h4 = 2 * c + hl4

        def pib(b, _):
            rs = pl.ds(pl.multiple_of(tid * RPT2 + b * RW, 8), RW)
            pltpu.sync_copy(h_all.at[h4, rs], kbuf)
            pltpu.sync_copy(kbuf, new_h.at[h4, rs])
            return 0
        lax.fori_loop(0, nb2, pib, 0)
    plsc.subcore_barrier()

    def zero_acc():
        def zb(b, _):
            rs = pl.ds(pl.multiple_of(tid * RPT + b * RW, 8), RW)
            pltpu.sync_copy(zrows, agg_sh.at[rs])
            return 0
        lax.fori_loop(0, nblocks, zb, 0)

    def fire_idx(eofs, ci):
        eb = pl.multiple_of(eofs + tid * EPT + ci * EC, 8)
        pltpu.async_copy(esrc.at[pl.ds(eb, EC)], idx_s, sis)
        pltpu.async_copy(edst.at[pl.ds(eb, EC)], idx_d, sid)

    def wait_idx(both):
        pltpu.make_async_copy(esrc.at[pl.ds(0, EC)], idx_s, sis).wait()
        if both:
            pltpu.make_async_copy(edst.at[pl.ds(0, EC)], idx_d, sid).wait()

    def build_idx(base_idx, base, idx_out):
        for u in range(5):
            sl = pl.ds(u * 16, 16)
            idx_out[sl] = base_idx[sl] + base

    def round0(eofs, kbase, qbase, vbase):
        fire_idx(eofs, 0)

        def chunk(ci, _):
            wait_idx(True)
            build_idx(idx_s, kbase, idx_k)
            build_idx(idx_d, qbase, idx_q)
            build_idx(idx_s, vbase, idx_v)
            build_idx(idx_d, 0, idx_sc)
            pltpu.async_copy(TKQ.at[idx_k], kbuf, sgk)
            pltpu.async_copy(TKQ.at[idx_q], qbuf, sgq)
            pltpu.async_copy(TV.at[idx_v], vbuf, sgv)

            @pl.when(ci + 1 < NCH)
            def _pf():
                fire_idx(eofs, ci + 1)

            pltpu.make_async_copy(TKQ.at[idx_k], kbuf, sgk).wait()
            pltpu.make_async_copy(TKQ.at[idx_q], qbuf, sgq).wait()
            pltpu.make_async_copy(TV.at[idx_v], vbuf, sgv).wait()

            def grp(g2, _g):
                pacc = jnp.zeros((16,), jnp.float32)
                for j2 in range(16):
                    j = g2 * 16 + j2
                    acc = jnp.zeros((16,), jnp.float32)
                    for u in range(12):
                        sl = pl.ds(u * 16, 16)
                        acc = acc + qbuf[j, sl] * kbuf[j, sl]
                    pv = jnp.exp(jnp.full((16,), jnp.sum(acc), jnp.float32))
                    pacc = pacc + pv * onehot[j2]
                    for u in range(6):
                        sl = pl.ds(u * 16, 16)
                        mbuf[j, sl] = vbuf[j, sl] * pv
                    mbuf[j, pl.ds(VH, 16)] = pv * e0mask
                ptile[pl.ds((ci * 5 + g2) * 16, 16)] = pacc
                return 0

            lax.fori_loop(0, 5, grp, 0)
            pltpu.sync_copy(mbuf, agg_sh.at[idx_sc], add=True)
            return 0

        lax.fori_loop(0, NCH, chunk, 0)

    def round1(eofs, vbase):
        fire_idx(eofs, 0)

        def chunk(ci, _):
            wait_idx(True)
            build_idx(idx_s, vbase, idx_v)
            build_idx(idx_d, 0, idx_sc)
            pltpu.async_copy(TV.at[idx_v], vbuf, sgv)

            @pl.when(ci + 1 < NCH)
            def _pf():
                fire_idx(eofs, ci + 1)

            pltpu.make_async_copy(TV.at[idx_v], vbuf, sgv).wait()

            def grp(g2, _g):
                pvec = ptile[pl.ds((ci * 5 + g2) * 16, 16)]
                for j2 in range(16):
                    j = g2 * 16 + j2
                    pv = jnp.full((16,), pvec[j2], jnp.float32)
                    for u in range(6):
                        sl = pl.ds(u * 16, 16)
                        mbuf[j, sl] = vbuf[j, sl] * pv
                    mbuf[j, pl.ds(VH, 16)] = pv * e0mask
                return 0

            lax.fori_loop(0, 5, grp, 0)
            pltpu.sync_copy(mbuf, agg_sh.at[idx_sc], add=True)
            return 0

        lax.fori_loop(0, NCH, chunk, 0)

    def finalize(head, nodeofs, half, scale):
        hs = pl.ds(half * VH, VH)

        def fb(b, _):
            r0 = pl.multiple_of(tid * RPT + b * RW, 8)
            rs = pl.ds(r0, RW)
            rs2 = pl.ds(pl.multiple_of(nodeofs + r0, 8), RW)
            pltpu.sync_copy(agg_sh.at[rs], mbuf)
            pltpu.sync_copy(new_h.at[head, rs2, hs], vbuf)

            def frow(j, _f):
                sv = jnp.full((16,), mbuf[j, pl.ds(VH, 16)][0], jnp.float32)
                rv = jnp.full((16,), scale, jnp.float32) / (sv + 1e-9)
                for u in range(6):
                    sl = pl.ds(u * 16, 16)
                    vbuf[j, sl] = mbuf[j, sl] * rv + vbuf[j, sl]
                return 0

            lax.fori_loop(0, RW, frow, 0)
            pltpu.sync_copy(vbuf, new_h.at[head, rs2, hs])
            return 0

        lax.fori_loop(0, nblocks, fb, 0)

    def rel_body(ri, _):
        hl = ri // 3
        rr = ri - hl * 3
        head = 2 * c + hl
        n = jnp.int32(NQ)
        toff = 2 * H * NQ
        kbase = jnp.where(rr == 0, toff + head * n,
                          jnp.where(rr == 1, head * n,
                                    toff + (H + head) * n))
        qbase = jnp.where(rr == 0, (H + head) * n, toff + (2 * H + head) * n)
        voff = 2 * H * NQ
        vbase = jnp.where(rr == 0, voff + head * 2 * n,
                          jnp.where(rr == 1, head * 2 * n,
                                    voff + (H + head) * 2 * n))
        eofs = jnp.where(rr == 0, E, jnp.where(rr == 1, 0, 2 * E))
        nodeofs = jnp.where(rr == 0, 0, NQ)
        scale = jnp.where(rr == 0, fone, jnp.float32(0.5))

        zero_acc()
        plsc.subcore_barrier()
        round0(eofs, kbase, qbase, vbase)
        plsc.subcore_barrier()
        finalize(head, nodeofs, 0, scale)
        zero_acc()
        plsc.subcore_barrier()
        round1(eofs, vbase + n)
        plsc.subcore_barrier()
        finalize(head, nodeofs, 1, scale)
        return 0

    lax.fori_loop(0, 6, rel_body, 0)


_SC_SCRATCH = [
    pltpu.VMEM_SHARED((NT, AW), jnp.float32),
    pltpu.VMEM((EC,), _i32),
    pltpu.VMEM((EC,), _i32),
    pltpu.VMEM((EC,), _i32),
    pltpu.VMEM((EC,), _i32),
    pltpu.VMEM((EC,), _i32),
    pltpu.VMEM((EC,), _i32),
    pltpu.VMEM((EC, DK), jnp.float32),
    pltpu.VMEM((EC, DK), jnp.float32),
    pltpu.VMEM((EC, VH), jnp.float32),
    pltpu.VMEM((EC, AW), jnp.float32),
    pltpu.VMEM((EPT,), jnp.float32),
    pltpu.SemaphoreType.DMA,
    pltpu.SemaphoreType.DMA,
    pltpu.SemaphoreType.DMA,
    pltpu.SemaphoreType.DMA,
    pltpu.SemaphoreType.DMA,
]


def _sc_layer(TKQ, TV, h_all, esrc, edst):
    zrows = jnp.zeros((RW, AW), jnp.float32)
    fn = pl.kernel(
        _sc_layer_kernel,
        out_type=jax.ShapeDtypeStruct((H, N2, DK), jnp.float32),
        mesh=plsc.VectorSubcoreMesh(core_axis_name="c", subcore_axis_name="s"),
        scratch_types=_SC_SCRATCH,
        compiler_params=pltpu.CompilerParams(use_tc_tiling_on_sc=False,
                                             needs_layout_passes=False),
    )
    return fn(TKQ, TV, h_all, esrc, edst, zrows)


def kernel(h_query, h_tag, give_src, give_dst, inter1_src, inter1_dst,
           inter2_src, inter2_dst, Wk, bk, Wq, bq, Wv, bv, rel_att, rel_msg,
           rel_pri, Wout, bout):
    scale = (rel_pri / SQRT_DK)[:, :, :, None, None]
    att_s = (rel_att * scale).reshape(L * R, H, DK, DK)
    msg = rel_msg.reshape(L * R, H, DK, DK)
    bd_att = _block_diag(att_s)
    bd_msg = _block_diag(msg)
    stype = jnp.array([0, 1, 1], jnp.int32)
    WkT = jnp.transpose(Wk, (0, 1, 3, 2))[:, stype]
    WvT = jnp.transpose(Wv, (0, 1, 3, 2))[:, stype]
    A = jnp.concatenate([WkT.reshape(L * R, D, D),
                         WvT.reshape(L * R, D, D)], axis=0)
    BD = jnp.concatenate([bd_att, bd_msg], axis=0)
    folded = _fold_mm(A, BD)
    WKf = folded[:L * R].reshape(L, R, D, D)
    WVf = folded[L * R:].reshape(L, R, D, D)
    bkr = bk[:, stype]
    bvr = bv[:, stype]
    bKf = jnp.einsum('lrd,lrde->lre', bkr, bd_att.reshape(L, R, D, D))
    bVf = jnp.einsum('lrd,lrde->lre', bvr, bd_msg.reshape(L, R, D, D))
    WqT = jnp.transpose(Wq, (0, 1, 3, 2))

    h_all = jnp.concatenate(
        [h_query.reshape(NQ, H, DK).transpose(1, 0, 2),
         h_tag.reshape(NT, H, DK).transpose(1, 0, 2)], axis=1)
    esrc = jnp.concatenate([give_src, inter1_src, inter2_src]).astype(_i32)
    edst = jnp.concatenate([give_dst, inter1_dst, inter2_dst]).astype(_i32)

    for l in range(L):
        Pkq = _proj_mm(
            h_all,
            jnp.stack([WKf[l, 0], WqT[l, 0], WKf[l, 1], WKf[l, 2], WqT[l, 1]]),
            jnp.stack([bKf[l, 0], bq[l, 0], bKf[l, 1], bKf[l, 2], bq[l, 1]]),
            splitg=2)
        Pv = _proj_mm_half(
            h_all,
            jnp.stack([WVf[l, 0], WVf[l, 1], WVf[l, 2]]),
            jnp.stack([bVf[l, 0], bVf[l, 1], bVf[l, 2]]),
            splitg=1)

        h_all = _sc_layer(Pkq.reshape(5 * H * NQ, DK),
                          Pv.reshape(3 * H * 2 * NQ, VH),
                          h_all, esrc, edst)

    out_q = _out_mm(h_all, Wout.T, bout, 0)
    out_t = _out_mm(h_all, Wout.T, bout, NQ // 1000)
    return (out_q, out_t)


# speedup vs baseline: 4.7205x; 1.1275x over previous
import functools

import jax
import jax.numpy as jnp
import numpy as np
from jax import lax
from jax.experimental import pallas as pl
from jax.experimental.pallas import tpu as pltpu
from jax.experimental.pallas import tpu_sc as plsc

NQ = 10000
NT = 10000
E = 32000
D = 768
H = 4
DK = D // H
L = 2
R = 3
SQRT_DK = float(np.sqrt(DK))

TM = 2000
_PREC = jax.lax.Precision.DEFAULT


def _fold_body(a_ref, b_ref, o_ref):
    o_ref[0] = jax.lax.dot_general(
        a_ref[0], b_ref[0], (((1,), (0,)), ((), ())),
        preferred_element_type=jnp.float32, precision=_PREC)


def _fold_mm(A, B):
    G = A.shape[0]
    return pl.pallas_call(
        _fold_body,
        grid=(G,),
        in_specs=[pl.BlockSpec((1, D, D), lambda g: (g, 0, 0)),
                  pl.BlockSpec((1, D, D), lambda g: (g, 0, 0))],
        out_specs=pl.BlockSpec((1, D, D), lambda g: (g, 0, 0)),
        out_shape=jax.ShapeDtypeStruct((G, D, D), jnp.float32),
    )(A, B)


def _proj_body(a_ref, b_ref, bias_ref, o_ref):
    h = pl.program_id(2)
    acc = bias_ref[0, h, :][None, :]
    for kh in range(H):
        acc = acc + jax.lax.dot_general(
            a_ref[kh], b_ref[0, pl.ds(kh * DK, DK), h, :],
            (((1,), (0,)), ((), ())),
            preferred_element_type=jnp.float32, precision=_PREC)
    o_ref[0, 0] = acc


def _proj_mm(A, B, bias, splitg):
    G = B.shape[0]
    Mt = NQ // TM
    Bh = B.reshape(G, D, H, DK)
    biash = bias.reshape(G, H, DK)

    def a_map(i, g, h):
        return (0, jnp.where(g >= splitg, i + Mt, i), 0)

    return pl.pallas_call(
        _proj_body,
        grid=(Mt, G, H),
        in_specs=[pl.BlockSpec((H, TM, DK), a_map),
                  pl.BlockSpec((1, D, H, DK), lambda i, g, h: (g, 0, 0, 0)),
                  pl.BlockSpec((1, H, DK), lambda i, g, h: (g, 0, 0))],
        out_specs=pl.BlockSpec((1, 1, TM, DK), lambda i, g, h: (g, h, i, 0)),
        out_shape=jax.ShapeDtypeStruct((G, H, NQ, DK), jnp.float32),
    )(A, Bh, biash)


def _projh_body(a_ref, b_ref, bias_ref, o_ref):
    h = pl.program_id(2)
    hh = pl.program_id(3)
    acc = bias_ref[0, h, hh, :][None, :]
    for kh in range(H):
        acc = acc + jax.lax.dot_general(
            a_ref[kh], b_ref[0, pl.ds(kh * DK, DK), h, hh, :],
            (((1,), (0,)), ((), ())),
            preferred_element_type=jnp.float32, precision=_PREC)
    o_ref[0, 0, 0] = acc


def _proj_mm_half(A, B, bias, splitg):
    G = B.shape[0]
    Mt = NQ // TM
    Bh = B.reshape(G, D, H, 2, VH)
    biash = bias.reshape(G, H, 2, VH)

    def a_map(i, g, h, hh):
        return (0, jnp.where(g >= splitg, i + Mt, i), 0)

    return pl.pallas_call(
        _projh_body,
        grid=(Mt, G, H, 2),
        in_specs=[pl.BlockSpec((H, TM, DK), a_map),
                  pl.BlockSpec((1, D, H, 2, VH),
                               lambda i, g, h, hh: (g, 0, 0, 0, 0)),
                  pl.BlockSpec((1, H, 2, VH),
                               lambda i, g, h, hh: (g, 0, 0, 0))],
        out_specs=pl.BlockSpec((1, 1, 1, TM, 96),
                               lambda i, g, h, hh: (g, h, hh, i, 0)),
        out_shape=jax.ShapeDtypeStruct((G, H, 2, NQ, 96), jnp.float32),
    )(A, Bh, biash)


def _out_body(a_ref, b_ref, bias_ref, o_ref):
    acc = bias_ref[...][None, :]
    for kh in range(H):
        acc = acc + jax.lax.dot_general(
            a_ref[kh], b_ref[pl.ds(kh * DK, DK), :], (((1,), (0,)), ((), ())),
            preferred_element_type=jnp.float32, precision=_PREC)
    o_ref[...] = acc


def _out_mm(A, B, bias, rofsb):
    TMO = 1000
    Mt = NQ // TMO
    return pl.pallas_call(
        _out_body,
        grid=(Mt,),
        in_specs=[pl.BlockSpec((H, TMO, DK), lambda i: (0, i + rofsb, 0)),
                  pl.BlockSpec((D, D), lambda i: (0, 0)),
                  pl.BlockSpec((D,), lambda i: (0,))],
        out_specs=pl.BlockSpec((TMO, D), lambda i: (i, 0)),
        out_shape=jax.ShapeDtypeStruct((NQ, D), jnp.float32),
    )(A, B, bias)


def _block_diag(rel):
    G = rel.shape[0]
    bd = jnp.zeros((G, H, DK, H, DK), jnp.float32)
    ih = jnp.arange(H)
    bd = bd.at[:, ih, :, ih, :].set(rel.transpose(1, 0, 2, 3))
    return bd.reshape(G, D, D)


EC = 40
EPT = E // 16
NCH = EPT // EC
GRP = (16, 16, 8)
PST = 48
RPT = 640
RW = 40
VH = 96
AW = 112

_i32 = jnp.int32


N2 = NQ + NT
RPT2 = 1280


def _sc_layer_kernel(TKQ, TV, h_all, esrc, edst, zrows,
                     new_h,
                     agg_sh, idx_s, idx_d, idx_k, idx_q, idx_v,
                     idx_scA, idx_scB,
                     kbufA, kbufB, qbufA, qbufB, vbufA, vbufB, mbuf, ptile,
                     sis, sid, sgk, sgq, sgv, ssc):
    c = lax.axis_index("c")
    tid = lax.axis_index("s")
    iota = lax.iota(_i32, 16)
    fone = jnp.float32(1.0)
    fzero = jnp.float32(0.0)
    e0mask = jnp.where(iota == 0, fone, fzero)
    onehot = [jnp.where(iota == j2, fone, fzero) for j2 in range(16)]

    nblocks = jnp.minimum(jnp.int32(NT) - tid * RPT, RPT) // RW

    nb2 = jnp.minimum(jnp.int32(N2) - tid * RPT2, RPT2) // RW
    for hl4 in range(2):
        h4 = 2 * c + hl4

        def pib(b, _):
            rs = pl.ds(pl.multiple_of(tid * RPT2 + b * RW, 8), RW)
            pltpu.sync_copy(h_all.at[h4, rs], kbufA)
            pltpu.sync_copy(kbufA, new_h.at[h4, rs])
            return 0
        lax.fori_loop(0, nb2, pib, 0)
    plsc.subcore_barrier()

    def zero_acc():
        def zb(b, _):
            rs = pl.ds(pl.multiple_of(tid * RPT + b * RW, 8), RW)
            pltpu.sync_copy(zrows, agg_sh.at[rs])
            return 0
        lax.fori_loop(0, nblocks, zb, 0)

    def fire_idx(eofs, ci):
        eb = pl.multiple_of(eofs + tid * EPT + ci * EC, 8)
        pltpu.async_copy(esrc.at[pl.ds(eb, EC)], idx_s, sis)
        pltpu.async_copy(edst.at[pl.ds(eb, EC)], idx_d, sid)

    def wait_idx():
        pltpu.make_async_copy(esrc.at[pl.ds(0, EC)], idx_s, sis).wait()
        pltpu.make_async_copy(edst.at[pl.ds(0, EC)], idx_d, sid).wait()

    def build_idx(base_idx, base, idx_out, ofs):
        for u in (0, 16, EC - 16):
            idx_out[pl.ds(ofs + u, 16)] = base_idx[pl.ds(u, 16)] + base

    def fire_gathers(kb, qb, vb, isc, with_kq, kbase, qbase, vbase):
        if with_kq:
            build_idx(idx_s, kbase, idx_k, 0)
            build_idx(idx_d, qbase, idx_q, 0)
            pltpu.async_copy(TKQ.at[idx_k], kb, sgk)
            pltpu.async_copy(TKQ.at[idx_q], qb, sgq)
        build_idx(idx_s, vbase, idx_v, 0)
        build_idx(idx_d, 0, isc, 0)
        pltpu.async_copy(TV.at[idx_v], vb, sgv)

    def wait_gathers(with_kq):
        if with_kq:
            pltpu.make_async_copy(TKQ.at[idx_k], kbufA, sgk).wait()
            pltpu.make_async_copy(TKQ.at[idx_q], qbufA, sgq).wait()
        pltpu.make_async_copy(TV.at[idx_v], vbufA, sgv).wait()

    def pipeline(eofs, with_kq, kbase, qbase, vbase, compute):
        banks = ((kbufA, qbufA, vbufA, idx_scA),
                 (kbufB, qbufB, vbufB, idx_scB))
        fire_idx(eofs, 0)
        wait_idx()
        fire_gathers(*banks[0], with_kq, kbase, qbase, vbase)
        fire_idx(eofs, 1)

        def pair(c2, _):
            for b in (0, 1):
                kb, qb, vb, isc = banks[b]
                ci = 2 * c2 + b
                wait_gathers(with_kq)

                if b == 0:
                    wait_idx()
                    fire_gathers(*banks[1], with_kq, kbase, qbase, vbase)

                    @pl.when(ci + 2 < NCH)
                    def _pf():
                        fire_idx(eofs, ci + 2)
                else:
                    @pl.when(ci + 1 < NCH)
                    def _nx():
                        wait_idx()
                        fire_gathers(*banks[0], with_kq,
                                     kbase, qbase, vbase)

                        @pl.when(ci + 2 < NCH)
                        def _pf():
                            fire_idx(eofs, ci + 2)

                compute(ci, kb, qb, vb)
                pltpu.sync_copy(mbuf, agg_sh.at[isc], add=True)
            return 0

        lax.fori_loop(0, NCH // 2, pair, 0)

    def round0(eofs, kbase, qbase, vbase):
        def compute(ci, kb, qb, vb):
            jofs = 0
            for g2, gsz in enumerate(GRP):
                pacc = jnp.zeros((16,), jnp.float32)
                for j2 in range(gsz):
                    j = jofs + j2
                    acc = jnp.zeros((16,), jnp.float32)
                    for u in range(12):
                        sl = pl.ds(u * 16, 16)
                        acc = acc + qb[j, sl] * kb[j, sl]
                    pv = jnp.exp(jnp.full((16,), jnp.sum(acc), jnp.float32))
                    pacc = pacc + pv * onehot[j2]
                    for u in range(6):
                        sl = pl.ds(u * 16, 16)
                        mbuf[j, sl] = vb[j, sl] * pv
                    mbuf[j, pl.ds(VH, 16)] = pv * e0mask
                ptile[pl.ds(ci * PST + g2 * 16, 16)] = pacc
                jofs += gsz

        pipeline(eofs, True, kbase, qbase, vbase, compute)

    def round1(eofs, vbase):
        def compute(ci, kb, qb, vb):
            jofs = 0
            for g2, gsz in enumerate(GRP):
                pvec = ptile[pl.ds(ci * PST + g2 * 16, 16)]
                for j2 in range(gsz):
                    j = jofs + j2
                    pv = jnp.full((16,), pvec[j2], jnp.float32)
                    for u in range(6):
                        sl = pl.ds(u * 16, 16)
                        mbuf[j, sl] = vb[j, sl] * pv
                    mbuf[j, pl.ds(VH, 16)] = pv * e0mask
                jofs += gsz

        pipeline(eofs, False, 0, 0, vbase, compute)

    def finalize(head, nodeofs, half, scale):
        hs = pl.ds(half * VH, VH)

        def fb(b, _):
            r0 = pl.multiple_of(tid * RPT + b * RW, 8)
            rs = pl.ds(r0, RW)
            rs2 = pl.ds(pl.multiple_of(nodeofs + r0, 8), RW)
            pltpu.sync_copy(agg_sh.at[rs], mbuf)
            pltpu.sync_copy(new_h.at[head, rs2, hs], vbufA)

            def frow(j, _f):
                sv = jnp.full((16,), mbuf[j, pl.ds(VH, 16)][0], jnp.float32)
                rv = jnp.full((16,), scale, jnp.float32) / (sv + 1e-9)
                for u in range(6):
                    sl = pl.ds(u * 16, 16)
                    vbufA[j, sl] = mbuf[j, sl] * rv + vbufA[j, sl]
                return 0

            lax.fori_loop(0, RW, frow, 0)
            pltpu.sync_copy(vbufA, new_h.at[head, rs2, hs])
            return 0

        lax.fori_loop(0, nblocks, fb, 0)

    def rel_body(ri, _):
        hl = ri // 3
        rr = ri - hl * 3
        head = 2 * c + hl
        n = jnp.int32(NQ)
        toff = 2 * H * NQ
        kbase = jnp.where(rr == 0, toff + head * n,
                          jnp.where(rr == 1, head * n,
                                    toff + (H + head) * n))
        qbase = jnp.where(rr == 0, (H + head) * n, toff + (2 * H + head) * n)
        voff = 2 * H * NQ
        vbase = jnp.where(rr == 0, voff + head * 2 * n,
                          jnp.where(rr == 1, head * 2 * n,
                                    voff + (H + head) * 2 * n))
        eofs = jnp.where(rr == 0, E, jnp.where(rr == 1, 0, 2 * E))
        nodeofs = jnp.where(rr == 0, 0, NQ)
        scale = jnp.where(rr == 0, fone, jnp.float32(0.5))

        zero_acc()
        plsc.subcore_barrier()
        round0(eofs, kbase, qbase, vbase)
        plsc.subcore_barrier()
        finalize(head, nodeofs, 0, scale)
        zero_acc()
        plsc.subcore_barrier()
        round1(eofs, vbase + n)
        plsc.subcore_barrier()
        finalize(head, nodeofs, 1, scale)
        return 0

    lax.fori_loop(0, 6, rel_body, 0)


_SC_SCRATCH = [
    pltpu.VMEM_SHARED((NT, AW), jnp.float32),
    pltpu.VMEM((EC,), _i32),
    pltpu.VMEM((EC,), _i32),
    pltpu.VMEM((EC,), _i32),
    pltpu.VMEM((EC,), _i32),
    pltpu.VMEM((EC,), _i32),
    pltpu.VMEM((EC,), _i32),
    pltpu.VMEM((EC,), _i32),
    pltpu.VMEM((EC, DK), jnp.float32),
    pltpu.VMEM((EC, DK), jnp.float32),
    pltpu.VMEM((EC, DK), jnp.float32),
    pltpu.VMEM((EC, DK), jnp.float32),
    pltpu.VMEM((EC, VH), jnp.float32),
    pltpu.VMEM((EC, VH), jnp.float32),
    pltpu.VMEM((EC, AW), jnp.float32),
    pltpu.VMEM((NCH * PST,), jnp.float32),
    pltpu.SemaphoreType.DMA,
    pltpu.SemaphoreType.DMA,
    pltpu.SemaphoreType.DMA,
    pltpu.SemaphoreType.DMA,
    pltpu.SemaphoreType.DMA,
    pltpu.SemaphoreType.DMA,
]


def _sc_layer(TKQ, TV, h_all, esrc, edst):
    zrows = jnp.zeros((RW, AW), jnp.float32)
    fn = pl.kernel(
        _sc_layer_kernel,
        out_type=jax.ShapeDtypeStruct((H, N2, DK), jnp.float32),
        mesh=plsc.VectorSubcoreMesh(core_axis_name="c", subcore_axis_name="s"),
        scratch_types=_SC_SCRATCH,
        compiler_params=pltpu.CompilerParams(use_tc_tiling_on_sc=False,
                                             needs_layout_passes=False),
    )
    return fn(TKQ, TV, h_all, esrc, edst, zrows)


def kernel(h_query, h_tag, give_src, give_dst, inter1_src, inter1_dst,
           inter2_src, inter2_dst, Wk, bk, Wq, bq, Wv, bv, rel_att, rel_msg,
           rel_pri, Wout, bout):
    scale = (rel_pri / SQRT_DK)[:, :, :, None, None]
    att_s = (rel_att * scale).reshape(L * R, H, DK, DK)
    msg = rel_msg.reshape(L * R, H, DK, DK)
    bd_att = _block_diag(att_s)
    bd_msg = _block_diag(msg)
    stype = jnp.array([0, 1, 1], jnp.int32)
    WkT = jnp.transpose(Wk, (0, 1, 3, 2))[:, stype]
    WvT = jnp.transpose(Wv, (0, 1, 3, 2))[:, stype]
    A = jnp.concatenate([WkT.reshape(L * R, D, D),
                         WvT.reshape(L * R, D, D)], axis=0)
    BD = jnp.concatenate([bd_att, bd_msg], axis=0)
    folded = _fold_mm(A, BD)
    WKf = folded[:L * R].reshape(L, R, D, D)
    WVf = folded[L * R:].reshape(L, R, D, D)
    bkr = bk[:, stype]
    bvr = bv[:, stype]
    bKf = jnp.einsum('lrd,lrde->lre', bkr, bd_att.reshape(L, R, D, D))
    bVf = jnp.einsum('lrd,lrde->lre', bvr, bd_msg.reshape(L, R, D, D))
    WqT = jnp.transpose(Wq, (0, 1, 3, 2))

    h_all = jnp.concatenate(
        [h_query.reshape(NQ, H, DK).transpose(1, 0, 2),
         h_tag.reshape(NT, H, DK).transpose(1, 0, 2)], axis=1)
    esrc = jnp.concatenate([give_src, inter1_src, inter2_src]).astype(_i32)
    edst = jnp.concatenate([give_dst, inter1_dst, inter2_dst]).astype(_i32)

    for l in range(L):
        Pkq = _proj_mm(
            h_all,
            jnp.stack([WKf[l, 0], WqT[l, 0], WKf[l, 1], WKf[l, 2], WqT[l, 1]]),
            jnp.stack([bKf[l, 0], bq[l, 0], bKf[l, 1], bKf[l, 2], bq[l, 1]]),
            splitg=2)
        Pv = _proj_mm_half(
            h_all,
            jnp.stack([WVf[l, 0], WVf[l, 1], WVf[l, 2]]),
            jnp.stack([bVf[l, 0], bVf[l, 1], bVf[l, 2]]),
            splitg=1)

        h_all = _sc_layer(Pkq.reshape(5 * H * NQ, DK),
                          Pv.reshape(3 * H * 2 * NQ, VH),
                          h_all, esrc, edst)

    out_q = _out_mm(h_all, Wout.T, bout, 0)
    out_t = _out_mm(h_all, Wout.T, bout, NQ // 1000)
    return (out_q, out_t)
